# Initial kernel scaffold; baseline (speedup 1.0000x reference)
import jax
import jax.numpy as jnp
from jax.experimental import pallas as pl


def kernel(x, pos, batch, enc_params, dec_params):
    raise NotImplementedError("write your pallas kernel here")


# speedup vs baseline: 4.8881x; 4.8881x over previous
import functools
import math

import jax
import jax.numpy as jnp
from jax import lax
from jax.experimental import pallas as pl
from jax.experimental.pallas import tpu as pltpu
from jax.experimental.pallas import tpu_sc as plsc

_F32 = jnp.float32
_INF = float("inf")


def _dot(a, b):
    return lax.dot_general(
        a, b, (((1,), (0,)), ((), ())),
        preferred_element_type=jnp.float32)


def _fps(pos_l, m):
    n = pos_l.shape[0]
    C = n // 8
    posP = jnp.pad(pos_l, ((0, 0), (0, 125)))
    X = jnp.concatenate([pos_l[:, 0].reshape(8, C),
                         pos_l[:, 1].reshape(8, C),
                         pos_l[:, 2].reshape(8, C)], axis=0)

    def kern(posP_ref, X_ref, psel_ref):
        xs = X_ref[0:8, :]
        ys = X_ref[8:16, :]
        zs = X_ref[16:24, :]
        psel_ref[0:1, :] = posP_ref[0:1, :]
        ir = lax.broadcasted_iota(jnp.int32, (8, C), 0)
        ic = lax.broadcasted_iota(jnp.int32, (8, C), 1)
        flat = ir * C + ic
        dmin0 = jnp.full((8, C), _INF, dtype=_F32)

        def body(i, carry):
            last, dmin = carry
            prow = posP_ref[pl.ds(last, 1), :]
            px = prow[0:1, 0:1]
            py = prow[0:1, 1:2]
            pz = prow[0:1, 2:3]
            dx = xs - px
            dy = ys - py
            dz = zs - pz
            d = dx * dx + dy * dy + dz * dz
            dmin = jnp.minimum(dmin, d)
            m0 = jnp.max(dmin)
            nxt = jnp.min(jnp.where(dmin == m0, flat, n)).astype(jnp.int32)
            psel_ref[pl.ds(i, 1), :] = posP_ref[pl.ds(nxt, 1), :]
            return (nxt, dmin)

        lax.fori_loop(1, m, body, (jnp.int32(0), dmin0))

    psel = pl.pallas_call(
        kern,
        out_shape=jax.ShapeDtypeStruct((m, 128), jnp.float32),
    )(posP, X)
    return psel[:, :3]


def _neighbor_topk(pos_src, pos_q, K, r=None):
    n = pos_src.shape[0]
    m = pos_q.shape[0]
    B = min(128, m)
    CH = min(n, 1024)
    nch = n // CH
    radius_mode = r is not None
    posT = jnp.pad(pos_src.T, ((0, 5), (0, 0)))
    pp8 = jnp.pad(jnp.sum(pos_src * pos_src, axis=1)[None, :],
                  ((0, 7), (0, 0)))
    qpad = jnp.pad(pos_q, ((0, 0), (0, 5)))
    r2 = float(r) * float(r) if radius_mode else None

    def kern(q_ref, posT_ref, pp_ref, cols_ref, aux_ref, work_ref, cont_ref):
        q = q_ref[...]
        qq = jnp.sum(q * q, axis=1, keepdims=True)
        for c in range(nch):
            sl = slice(c * CH, (c + 1) * CH)
            dd = _dot(q, posT_ref[:, sl])
            d2 = jnp.maximum(qq + pp_ref[0:1, sl] - 2.0 * dd, 0.0)
            if radius_mode:
                d2 = jnp.where(d2 <= r2, d2, _INF)
            work_ref[:, sl] = d2
        cols_ref[...] = jnp.zeros((B, K), jnp.int32)
        if radius_mode:
            aux_ref[...] = jnp.zeros((B, K), jnp.int32)
        cont_ref[0] = 1
        lane = lax.broadcasted_iota(jnp.int32, (B, CH), 1)
        for k in range(K):
            @pl.when(cont_ref[0] != 0)
            def _step(k=k):
                v = jnp.full((B, 1), _INF, _F32)
                for c in range(nch):
                    sl = slice(c * CH, (c + 1) * CH)
                    v = jnp.minimum(
                        v, jnp.min(work_ref[:, sl], axis=1, keepdims=True))
                cidx = jnp.full((B, 1), n, jnp.int32)
                for c in range(nch):
                    sl = slice(c * CH, (c + 1) * CH)
                    cand = jnp.where(work_ref[:, sl] == v, lane + c * CH, n)
                    cidx = jnp.minimum(
                        cidx, jnp.min(cand, axis=1, keepdims=True))
                ok = v < _INF
                if radius_mode:
                    cols_ref[:, k:k + 1] = jnp.where(ok, cidx, 0)
                    aux_ref[:, k:k + 1] = ok.astype(jnp.int32)
                    cont_ref[0] = jnp.any(ok).astype(jnp.int32)
                else:
                    cols_ref[:, k:k + 1] = cidx
                    aux_ref[:, k:k + 1] = v
                for c in range(nch):
                    sl = slice(c * CH, (c + 1) * CH)
                    wch = work_ref[:, sl]
                    work_ref[:, sl] = jnp.where(
                        lane + c * CH == cidx, _INF, wch)

    aux_dtype = jnp.int32 if radius_mode else jnp.float32
    cols, aux = pl.pallas_call(
        kern,
        grid=(m // B,),
        in_specs=[
            pl.BlockSpec((B, 8), lambda i: (i, 0)),
            pl.BlockSpec((8, n), lambda i: (0, 0)),
            pl.BlockSpec((8, n), lambda i: (0, 0)),
        ],
        out_specs=[
            pl.BlockSpec((B, K), lambda i: (i, 0)),
            pl.BlockSpec((B, K), lambda i: (i, 0)),
        ],
        out_shape=[
            jax.ShapeDtypeStruct((m, K), jnp.int32),
            jax.ShapeDtypeStruct((m, K), aux_dtype),
        ],
        scratch_shapes=[
            pltpu.VMEM((B, n), jnp.float32),
            pltpu.SMEM((1,), jnp.int32),
        ],
    )(qpad, posT, pp8)
    return cols, aux


def _sc_gather(table, idx):
    V, D = table.shape
    B = idx.shape[0]
    info = plsc.get_sparse_core_info()
    NC, NS = info.num_cores, info.num_subcores
    NW = NC * NS
    assert B % NW == 0 and D % 16 == 0
    bpw = B // NW
    chunk = None
    for c in range(min(bpw, 128), 0, -1):
        if bpw % c == 0 and c % 8 == 0:
            chunk = c
            break
    assert chunk is not None, (B, bpw)
    nch = bpw // chunk
    mesh = plsc.VectorSubcoreMesh(core_axis_name="c", subcore_axis_name="s")

    @functools.partial(
        pl.kernel, mesh=mesh,
        out_type=jax.ShapeDtypeStruct((B, D), jnp.float32),
        scratch_types=[
            pltpu.VMEM((chunk,), jnp.int32),
            pltpu.VMEM((chunk, D), jnp.float32),
            pltpu.SemaphoreType.DMA,
        ],
    )
    def k(table_hbm, idx_hbm, out_hbm, idx_v, rows_v, sem):
        wid = lax.axis_index("s") * NC + lax.axis_index("c")
        base = wid * bpw

        def body(i, carry):
            start = base + i * chunk
            pltpu.sync_copy(idx_hbm.at[pl.ds(start, chunk)], idx_v)
            pltpu.async_copy(table_hbm.at[idx_v], rows_v, sem).wait()
            pltpu.sync_copy(rows_v, out_hbm.at[pl.ds(start, chunk)])
            return carry

        lax.fori_loop(0, nch, body, 0)

    return k(table, idx)


def _mlp(A, Ws, bs, bm=512):
    nrows, D = A.shape
    nl = len(Ws)
    bm = min(bm, nrows)
    Fout = Ws[-1].shape[1]

    def kern(a_ref, *refs):
        out_ref = refs[-1]
        h = a_ref[...]
        for i in range(nl):
            h = _dot(h, refs[2 * i][...]) + refs[2 * i + 1][...]
            if i < nl - 1:
                h = jnp.maximum(h, 0.0)
        out_ref[...] = h

    in_specs = [pl.BlockSpec((bm, D), lambda i: (i, 0))]
    args = [A]
    for W, b in zip(Ws, bs):
        in_specs.append(pl.BlockSpec(W.shape, lambda i: (0, 0)))
        in_specs.append(pl.BlockSpec((1, b.shape[0]), lambda i: (0, 0)))
        args.append(W)
        args.append(b.reshape(1, -1))
    return pl.pallas_call(
        kern,
        grid=(nrows // bm,),
        in_specs=in_specs,
        out_specs=pl.BlockSpec((bm, Fout), lambda i: (i, 0)),
        out_shape=jax.ShapeDtypeStruct((nrows, Fout), jnp.float32),
    )(*args)


def _sa_tail(g, qpad, vneg8, W1r, b1, W2, b2, W3, b3, m, K):
    F1 = W2.shape[0]
    F3 = W3.shape[1]
    Bq = min(128, m)
    nq = m // Bq
    NEG = float("-inf")

    def kern(g_ref, q_ref, vn_ref, w1_ref, b1_ref, w2_ref, b2_ref,
             w3_ref, b3_ref, out_ref):
        j = pl.program_id(1)
        q = q_ref[...]
        cterm = _dot(q, w1_ref[...]) - b1_ref[...]
        h1 = jnp.maximum(g_ref[...] - cterm, 0.0)
        h2 = jnp.maximum(_dot(h1, w2_ref[...]) + b2_ref[...], 0.0)
        h3 = _dot(h2, w3_ref[...]) + b3_ref[...]
        hm = h3 + vn_ref[...][:, 0:1]

        @pl.when(j == 0)
        def _():
            out_ref[...] = hm

        @pl.when(j != 0)
        def _():
            out_ref[...] = jnp.maximum(out_ref[...], hm)

        @pl.when(j == K - 1)
        def _():
            o = out_ref[...]
            out_ref[...] = jnp.where(o == NEG, 0.0, o)

    return pl.pallas_call(
        kern,
        grid=(nq, K),
        in_specs=[
            pl.BlockSpec((Bq, F1), lambda i, j: (j * nq + i, 0)),
            pl.BlockSpec((Bq, 8), lambda i, j: (i, 0)),
            pl.BlockSpec((Bq, 8), lambda i, j: (j * nq + i, 0)),
            pl.BlockSpec(W1r.shape, lambda i, j: (0, 0)),
            pl.BlockSpec((1, F1), lambda i, j: (0, 0)),
            pl.BlockSpec(W2.shape, lambda i, j: (0, 0)),
            pl.BlockSpec((1, W2.shape[1]), lambda i, j: (0, 0)),
            pl.BlockSpec(W3.shape, lambda i, j: (0, 0)),
            pl.BlockSpec((1, F3), lambda i, j: (0, 0)),
        ],
        out_specs=pl.BlockSpec((Bq, F3), lambda i, j: (i, 0)),
        out_shape=jax.ShapeDtypeStruct((m, F3), jnp.float32),
    )(g, qpad, vneg8, W1r, b1.reshape(1, -1), W2, b2.reshape(1, -1),
      W3, b3.reshape(1, -1))


def _interp3(g0, g1, g2, d2sel8, bm=512):
    m, F = g0.shape
    bm = min(bm, m)

    def kern(g0_ref, g1_ref, g2_ref, d_ref, out_ref):
        d = d_ref[...]
        w0 = 1.0 / jnp.maximum(d[:, 0:1], 1e-16)
        w1 = 1.0 / jnp.maximum(d[:, 1:2], 1e-16)
        w2 = 1.0 / jnp.maximum(d[:, 2:3], 1e-16)
        num = g0_ref[...] * w0 + g1_ref[...] * w1 + g2_ref[...] * w2
        out_ref[...] = num / (w0 + w1 + w2)

    gspec = pl.BlockSpec((bm, F), lambda i: (i, 0))
    return pl.pallas_call(
        kern,
        grid=(m // bm,),
        in_specs=[gspec, gspec, gspec, pl.BlockSpec((bm, 8), lambda i: (i, 0))],
        out_specs=gspec,
        out_shape=jax.ShapeDtypeStruct((m, F), jnp.float32),
    )(g0, g1, g2, d2sel8)


def _sa_module(x_l, pos_l, r, K, params):
    n = pos_l.shape[0]
    m = n // 2
    (W1, b1), (W2, b2), (W3, b3) = params
    F = x_l.shape[1]
    F1 = W1.shape[1]
    F1p = -(-F1 // 128) * 128
    if F1p != F1:
        W1 = jnp.pad(W1, ((0, 0), (0, F1p - F1)))
        b1 = jnp.pad(b1, (0, F1p - F1))
        W2 = jnp.pad(W2, ((0, F1p - F1), (0, 0)))
    pos_q = _fps(pos_l, m)
    cols, valid = _neighbor_topk(pos_l, pos_q, K, r=r)
    A = jnp.concatenate([x_l, pos_l], axis=1)
    u = _mlp(A, [W1], [jnp.zeros_like(b1)])
    idx_flat = cols.T.reshape(-1)
    g = _sc_gather(u, idx_flat)
    vneg = jnp.where(valid != 0, 0.0, -jnp.inf).astype(jnp.float32)
    vneg8 = jnp.broadcast_to(vneg.T.reshape(-1, 1), (K * m, 8))
    W1r = jnp.pad(W1[F:F + 3], ((0, 5), (0, 0)))
    qpad = jnp.pad(pos_q, ((0, 0), (0, 5)))
    out = _sa_tail(g, qpad, vneg8, W1r, b1, W2, b2, W3, b3, m, K)
    return out, pos_q


def _fp_module(xc, pos_c, x_skip, pos_s, k, params):
    Ws = [w for (w, _) in params]
    bs = [b for (_, b) in params]
    if k == 1:
        cols, _ = _neighbor_topk(pos_c, pos_s, 1, r=None)
        xi = _sc_gather(xc, cols[:, 0])
    else:
        cols, d2sel = _neighbor_topk(pos_c, pos_s, k, r=None)
        ms = pos_s.shape[0]
        g = _sc_gather(xc, cols.T.reshape(-1))
        F = xc.shape[1]
        d2sel8 = jnp.pad(d2sel, ((0, 0), (0, 8 - k)))
        xi = _interp3(g[0:ms], g[ms:2 * ms], g[2 * ms:3 * ms], d2sel8)
    h = jnp.concatenate([xi, x_skip], axis=1)
    return _mlp(h, Ws, bs)


def kernel(x, pos, batch, enc_params, dec_params):
    x1, p1 = _sa_module(x, pos, 0.05, 32, enc_params[0])
    x2, p2 = _sa_module(x1, p1, 0.1, 32, enc_params[1])
    x3, p3 = _sa_module(x2, p2, 0.2, 32, enc_params[2])
    x4, p4 = _sa_module(x3, p3, 0.4, 32, enc_params[3])
    f4 = _fp_module(x4, p4, x3, p3, 1, dec_params[0])
    f3 = _fp_module(f4, p3, x2, p2, 3, dec_params[1])
    f2 = _fp_module(f3, p2, x1, p1, 3, dec_params[2])
    f1 = _fp_module(f2, p1, x, pos, 3, dec_params[3])
    return (f1, pos, batch)


# speedup vs baseline: 7.2233x; 1.4777x over previous
import functools
import math

import jax
import jax.numpy as jnp
from jax import lax
from jax.experimental import pallas as pl
from jax.experimental.pallas import tpu as pltpu
from jax.experimental.pallas import tpu_sc as plsc

_F32 = jnp.float32
_INF = float("inf")


def _dot(a, b):
    return lax.dot_general(
        a, b, (((1,), (0,)), ((), ())),
        preferred_element_type=jnp.float32)


def _fps(pos_l, m):
    n = pos_l.shape[0]
    C = n // 8
    posP = jnp.pad(pos_l, ((0, 0), (0, 125)))
    X = jnp.concatenate([pos_l[:, 0].reshape(8, C),
                         pos_l[:, 1].reshape(8, C),
                         pos_l[:, 2].reshape(8, C)], axis=0)

    def kern(posP_ref, X_ref, psel_ref):
        xs = X_ref[0:8, :]
        ys = X_ref[8:16, :]
        zs = X_ref[16:24, :]
        psel_ref[0:1, :] = posP_ref[0:1, :]
        ir = lax.broadcasted_iota(jnp.int32, (8, C), 0)
        ic = lax.broadcasted_iota(jnp.int32, (8, C), 1)
        flat = ir * C + ic
        dmin0 = jnp.full((8, C), _INF, dtype=_F32)

        def body(i, carry):
            last, dmin = carry
            prow = posP_ref[pl.ds(last, 1), :]
            px = prow[0:1, 0:1]
            py = prow[0:1, 1:2]
            pz = prow[0:1, 2:3]
            dx = xs - px
            dy = ys - py
            dz = zs - pz
            d = dx * dx + dy * dy + dz * dz
            dmin = jnp.minimum(dmin, d)
            m0 = jnp.max(dmin)
            nxt = jnp.min(jnp.where(dmin == m0, flat, n)).astype(jnp.int32)
            psel_ref[pl.ds(i, 1), :] = posP_ref[pl.ds(nxt, 1), :]
            return (nxt, dmin)

        lax.fori_loop(1, m, body, (jnp.int32(0), dmin0))

    psel = pl.pallas_call(
        kern,
        out_shape=jax.ShapeDtypeStruct((m, 128), jnp.float32),
    )(posP, X)
    return psel[:, :3]


def _neighbor_topk(pos_src, pos_q, K, r=None):
    n = pos_src.shape[0]
    m = pos_q.shape[0]
    B = min(128, m)
    CH = min(n, 1024)
    nch = n // CH
    radius_mode = r is not None
    posT = jnp.pad(pos_src.T, ((0, 5), (0, 0)))
    pp8 = jnp.pad(jnp.sum(pos_src * pos_src, axis=1)[None, :],
                  ((0, 7), (0, 0)))
    qpad = jnp.pad(pos_q, ((0, 0), (0, 5)))
    r2 = float(r) * float(r) if radius_mode else None

    def kern(q_ref, posT_ref, pp_ref, cols_ref, aux_ref, work_ref, cont_ref):
        q = q_ref[...]
        qq = jnp.sum(q * q, axis=1, keepdims=True)
        for c in range(nch):
            sl = slice(c * CH, (c + 1) * CH)
            dd = _dot(q, posT_ref[:, sl])
            d2 = jnp.maximum(qq + pp_ref[0:1, sl] - 2.0 * dd, 0.0)
            if radius_mode:
                d2 = jnp.where(d2 <= r2, d2, _INF)
            work_ref[:, sl] = d2
        cols_ref[...] = jnp.zeros((B, K), jnp.int32)
        if radius_mode:
            aux_ref[...] = jnp.zeros((B, K), jnp.int32)
        cont_ref[0] = 1
        lane = lax.broadcasted_iota(jnp.int32, (B, CH), 1)
        for k in range(K):
            @pl.when(cont_ref[0] != 0)
            def _step(k=k):
                v = jnp.full((B, 1), _INF, _F32)
                for c in range(nch):
                    sl = slice(c * CH, (c + 1) * CH)
                    v = jnp.minimum(
                        v, jnp.min(work_ref[:, sl], axis=1, keepdims=True))
                cidx = jnp.full((B, 1), n, jnp.int32)
                for c in range(nch):
                    sl = slice(c * CH, (c + 1) * CH)
                    cand = jnp.where(work_ref[:, sl] == v, lane + c * CH, n)
                    cidx = jnp.minimum(
                        cidx, jnp.min(cand, axis=1, keepdims=True))
                ok = v < _INF
                if radius_mode:
                    cols_ref[:, k:k + 1] = jnp.where(ok, cidx, 0)
                    aux_ref[:, k:k + 1] = ok.astype(jnp.int32)
                    cont_ref[0] = jnp.any(ok).astype(jnp.int32)
                else:
                    cols_ref[:, k:k + 1] = cidx
                    aux_ref[:, k:k + 1] = v
                for c in range(nch):
                    sl = slice(c * CH, (c + 1) * CH)
                    wch = work_ref[:, sl]
                    work_ref[:, sl] = jnp.where(
                        lane + c * CH == cidx, _INF, wch)

    aux_dtype = jnp.int32 if radius_mode else jnp.float32
    cols, aux = pl.pallas_call(
        kern,
        grid=(m // B,),
        in_specs=[
            pl.BlockSpec((B, 8), lambda i: (i, 0)),
            pl.BlockSpec((8, n), lambda i: (0, 0)),
            pl.BlockSpec((8, n), lambda i: (0, 0)),
        ],
        out_specs=[
            pl.BlockSpec((B, K), lambda i: (i, 0)),
            pl.BlockSpec((B, K), lambda i: (i, 0)),
        ],
        out_shape=[
            jax.ShapeDtypeStruct((m, K), jnp.int32),
            jax.ShapeDtypeStruct((m, K), aux_dtype),
        ],
        scratch_shapes=[
            pltpu.VMEM((B, n), jnp.float32),
            pltpu.SMEM((1,), jnp.int32),
        ],
    )(qpad, posT, pp8)
    return cols, aux


def _sc_gather(table, idx):
    V, D = table.shape
    B = idx.shape[0]
    info = plsc.get_sparse_core_info()
    NC, NS = info.num_cores, info.num_subcores
    NW = NC * NS
    assert B % NW == 0 and D % 16 == 0
    bpw = B // NW
    chunk = None
    for c in range(min(bpw, 128), 0, -1):
        if bpw % c == 0 and c % 8 == 0:
            chunk = c
            break
    assert chunk is not None, (B, bpw)
    nch = bpw // chunk
    mesh = plsc.VectorSubcoreMesh(core_axis_name="c", subcore_axis_name="s")

    @functools.partial(
        pl.kernel, mesh=mesh,
        out_type=jax.ShapeDtypeStruct((B, D), jnp.float32),
        scratch_types=[
            pltpu.VMEM((chunk,), jnp.int32),
            pltpu.VMEM((chunk, D), jnp.float32),
            pltpu.SemaphoreType.DMA,
        ],
    )
    def k(table_hbm, idx_hbm, out_hbm, idx_v, rows_v, sem):
        wid = lax.axis_index("s") * NC + lax.axis_index("c")
        base = wid * bpw

        def body(i, carry):
            start = base + i * chunk
            pltpu.sync_copy(idx_hbm.at[pl.ds(start, chunk)], idx_v)
            pltpu.async_copy(table_hbm.at[idx_v], rows_v, sem).wait()
            pltpu.sync_copy(rows_v, out_hbm.at[pl.ds(start, chunk)])
            return carry

        lax.fori_loop(0, nch, body, 0)

    return k(table, idx)


def _mlp(A, Ws, bs, bm=512):
    nrows, D = A.shape
    nl = len(Ws)
    bm = min(bm, nrows)
    Fout = Ws[-1].shape[1]

    def kern(a_ref, *refs):
        out_ref = refs[-1]
        h = a_ref[...]
        for i in range(nl):
            h = _dot(h, refs[2 * i][...]) + refs[2 * i + 1][...]
            if i < nl - 1:
                h = jnp.maximum(h, 0.0)
        out_ref[...] = h

    in_specs = [pl.BlockSpec((bm, D), lambda i: (i, 0))]
    args = [A]
    for W, b in zip(Ws, bs):
        in_specs.append(pl.BlockSpec(W.shape, lambda i: (0, 0)))
        in_specs.append(pl.BlockSpec((1, b.shape[0]), lambda i: (0, 0)))
        args.append(W)
        args.append(b.reshape(1, -1))
    return pl.pallas_call(
        kern,
        grid=(nrows // bm,),
        in_specs=in_specs,
        out_specs=pl.BlockSpec((bm, Fout), lambda i: (i, 0)),
        out_shape=jax.ShapeDtypeStruct((nrows, Fout), jnp.float32),
    )(*args)


def _sa_tail(g, qpad, vneg8, W1r, b1, W2, b2, W3, b3, m, K):
    F1 = W2.shape[0]
    F3 = W3.shape[1]
    Bq = min(128, m)
    nq = m // Bq
    NEG = float("-inf")

    def kern(g_ref, q_ref, vn_ref, w1_ref, b1_ref, w2_ref, b2_ref,
             w3_ref, b3_ref, out_ref):
        j = pl.program_id(1)
        q = q_ref[...]
        cterm = _dot(q, w1_ref[...]) - b1_ref[...]
        h1 = jnp.maximum(g_ref[...] - cterm, 0.0)
        h2 = jnp.maximum(_dot(h1, w2_ref[...]) + b2_ref[...], 0.0)
        h3 = _dot(h2, w3_ref[...]) + b3_ref[...]
        hm = h3 + vn_ref[...][:, 0:1]

        @pl.when(j == 0)
        def _():
            out_ref[...] = hm

        @pl.when(j != 0)
        def _():
            out_ref[...] = jnp.maximum(out_ref[...], hm)

        @pl.when(j == K - 1)
        def _():
            o = out_ref[...]
            out_ref[...] = jnp.where(o == NEG, 0.0, o)

    return pl.pallas_call(
        kern,
        grid=(nq, K),
        in_specs=[
            pl.BlockSpec((Bq, F1), lambda i, j: (j * nq + i, 0)),
            pl.BlockSpec((Bq, 8), lambda i, j: (i, 0)),
            pl.BlockSpec((Bq, 8), lambda i, j: (j * nq + i, 0)),
            pl.BlockSpec(W1r.shape, lambda i, j: (0, 0)),
            pl.BlockSpec((1, F1), lambda i, j: (0, 0)),
            pl.BlockSpec(W2.shape, lambda i, j: (0, 0)),
            pl.BlockSpec((1, W2.shape[1]), lambda i, j: (0, 0)),
            pl.BlockSpec(W3.shape, lambda i, j: (0, 0)),
            pl.BlockSpec((1, F3), lambda i, j: (0, 0)),
        ],
        out_specs=pl.BlockSpec((Bq, F3), lambda i, j: (i, 0)),
        out_shape=jax.ShapeDtypeStruct((m, F3), jnp.float32),
    )(g, qpad, vneg8, W1r, b1.reshape(1, -1), W2, b2.reshape(1, -1),
      W3, b3.reshape(1, -1))


def _interp3(g0, g1, g2, d2sel8, bm=512):
    m, F = g0.shape
    bm = min(bm, m)

    def kern(g0_ref, g1_ref, g2_ref, d_ref, out_ref):
        d = d_ref[...]
        w0 = 1.0 / jnp.maximum(d[:, 0:1], 1e-16)
        w1 = 1.0 / jnp.maximum(d[:, 1:2], 1e-16)
        w2 = 1.0 / jnp.maximum(d[:, 2:3], 1e-16)
        num = g0_ref[...] * w0 + g1_ref[...] * w1 + g2_ref[...] * w2
        out_ref[...] = num / (w0 + w1 + w2)

    gspec = pl.BlockSpec((bm, F), lambda i: (i, 0))
    return pl.pallas_call(
        kern,
        grid=(m // bm,),
        in_specs=[gspec, gspec, gspec, pl.BlockSpec((bm, 8), lambda i: (i, 0))],
        out_specs=gspec,
        out_shape=jax.ShapeDtypeStruct((m, F), jnp.float32),
    )(g0, g1, g2, d2sel8)


def _sa_module(x_l, pos_l, r, K, params):
    n = pos_l.shape[0]
    m = n // 2
    (W1, b1), (W2, b2), (W3, b3) = params
    F = x_l.shape[1]
    F1 = W1.shape[1]
    F1p = -(-F1 // 128) * 128
    if F1p != F1:
        W1 = jnp.pad(W1, ((0, 0), (0, F1p - F1)))
        b1 = jnp.pad(b1, (0, F1p - F1))
        W2 = jnp.pad(W2, ((0, F1p - F1), (0, 0)))
    pos_q = _fps(pos_l, m)
    cols, valid = _neighbor_topk(pos_l, pos_q, K, r=r)
    A = jnp.concatenate([x_l, pos_l], axis=1)
    u = _mlp(A, [W1], [jnp.zeros_like(b1)])
    idx_flat = cols.T.reshape(-1)
    vflat = valid.T.reshape(-1)
    idx_flat = jnp.where(vflat != 0, idx_flat,
                         jnp.arange(K * m, dtype=jnp.int32) % n)
    g = _sc_gather(u, idx_flat)
    vneg = jnp.where(valid != 0, 0.0, -jnp.inf).astype(jnp.float32)
    vneg8 = jnp.broadcast_to(vneg.T.reshape(-1, 1), (K * m, 8))
    W1r = jnp.pad(W1[F:F + 3], ((0, 5), (0, 0)))
    qpad = jnp.pad(pos_q, ((0, 0), (0, 5)))
    out = _sa_tail(g, qpad, vneg8, W1r, b1, W2, b2, W3, b3, m, K)
    return out, pos_q


def _fp_module(xc, pos_c, x_skip, pos_s, k, params):
    Ws = [w for (w, _) in params]
    bs = [b for (_, b) in params]
    if k == 1:
        cols, _ = _neighbor_topk(pos_c, pos_s, 1, r=None)
        xi = _sc_gather(xc, cols[:, 0])
    else:
        cols, d2sel = _neighbor_topk(pos_c, pos_s, k, r=None)
        ms = pos_s.shape[0]
        g = _sc_gather(xc, cols.T.reshape(-1))
        F = xc.shape[1]
        d2sel8 = jnp.pad(d2sel, ((0, 0), (0, 8 - k)))
        xi = _interp3(g[0:ms], g[ms:2 * ms], g[2 * ms:3 * ms], d2sel8)
    h = jnp.concatenate([xi, x_skip], axis=1)
    return _mlp(h, Ws, bs)


def kernel(x, pos, batch, enc_params, dec_params):
    x1, p1 = _sa_module(x, pos, 0.05, 32, enc_params[0])
    x2, p2 = _sa_module(x1, p1, 0.1, 32, enc_params[1])
    x3, p3 = _sa_module(x2, p2, 0.2, 32, enc_params[2])
    x4, p4 = _sa_module(x3, p3, 0.4, 32, enc_params[3])
    f4 = _fp_module(x4, p4, x3, p3, 1, dec_params[0])
    f3 = _fp_module(f4, p3, x2, p2, 3, dec_params[1])
    f2 = _fp_module(f3, p2, x1, p1, 3, dec_params[2])
    f1 = _fp_module(f2, p1, x, pos, 3, dec_params[3])
    return (f1, pos, batch)
